# baseline (speedup 1.0000x reference)
import functools
import math

import jax
import jax.numpy as jnp
from jax import lax
from jax.experimental import pallas as pl
from jax.experimental.pallas import tpu as pltpu
from jax.experimental.pallas import tpu_sc as plsc

N = 10000
E = 320000
D_E = 16
D = 128

NC = 2
NS = 16
NW = NC * NS
CHUNK = 128
CH_PER_W = 80
E_PAD = NW * CH_PER_W * CHUNK
N_PAD = 10240
ROWS_PER_TILE = N_PAD // NS

_HI = jax.lax.Precision.HIGHEST


def _seg_body(ef_hbm, dst_hbm, ones_hbm, zeros_hbm,
              psums_hbm, pcnts_hbm,
              idx_v, efb_v, ones_v, zeros_v, tmp_v, sums_sp, cnts_sp, sem):
    cid = lax.axis_index("c")
    sid = lax.axis_index("s")
    wid = sid * NC + cid
    r0 = sid * ROWS_PER_TILE

    pltpu.sync_copy(zeros_hbm, zeros_v)
    pltpu.sync_copy(zeros_v, sums_sp.at[pl.ds(r0, ROWS_PER_TILE)])
    pltpu.sync_copy(zeros_v, cnts_sp.at[pl.ds(r0, ROWS_PER_TILE)])
    pltpu.sync_copy(ones_hbm, ones_v)
    pltpu.sync_copy(dst_hbm.at[pl.ds(wid * CH_PER_W, CH_PER_W)], idx_v)
    plsc.subcore_barrier()

    def step(b, carry):
        base = (wid * CH_PER_W + b) * CHUNK
        pltpu.sync_copy(ef_hbm.at[pl.ds(base, CHUNK)], efb_v)
        pltpu.sync_copy(efb_v, sums_sp.at[idx_v.at[b]], add=True)
        pltpu.sync_copy(ones_v, cnts_sp.at[idx_v.at[b]], add=True)
        return carry

    lax.fori_loop(0, CH_PER_W, step, 0)
    plsc.subcore_barrier()

    pltpu.sync_copy(sums_sp.at[pl.ds(r0, ROWS_PER_TILE)], tmp_v)
    pltpu.sync_copy(tmp_v, psums_hbm.at[cid].at[pl.ds(r0, ROWS_PER_TILE)])
    pltpu.sync_copy(cnts_sp.at[pl.ds(r0, ROWS_PER_TILE)], tmp_v)
    pltpu.sync_copy(tmp_v, pcnts_hbm.at[cid].at[pl.ds(r0, ROWS_PER_TILE)])


_seg_kernel = functools.partial(
    pl.kernel,
    out_type=[
        jax.ShapeDtypeStruct((NC, N_PAD, D_E), jnp.float32),
        jax.ShapeDtypeStruct((NC, N_PAD, D_E), jnp.float32),
    ],
    mesh=plsc.VectorSubcoreMesh(core_axis_name="c", subcore_axis_name="s"),
    scratch_types=[
        pltpu.VMEM((CH_PER_W, CHUNK), jnp.int32),
        pltpu.VMEM((CHUNK, D_E), jnp.float32),
        pltpu.VMEM((CHUNK, D_E), jnp.float32),
        pltpu.VMEM((ROWS_PER_TILE, D_E), jnp.float32),
        pltpu.VMEM((ROWS_PER_TILE, D_E), jnp.float32),
        pltpu.VMEM_SHARED((N_PAD, D_E), jnp.float32),
        pltpu.VMEM_SHARED((N_PAD, D_E), jnp.float32),
        pltpu.SemaphoreType.DMA,
    ],
    compiler_params=pltpu.CompilerParams(use_tc_tiling_on_sc=False),
)


RB = 1024


def _node_body(nf_ref, wn_ref, bn_ref, psums_ref, pcnts_ref,
               we_ref, wd_ref, bd_ref, be_ref,
               nf3_ref, p_ref, bnn_ref, wc_ref):
    sums = psums_ref[0] + psums_ref[1]
    cnt = pcnts_ref[0, :, 0:1] + pcnts_ref[1, :, 0:1]
    nb = jnp.dot(sums, we_ref[...], precision=_HI,
                 preferred_element_type=jnp.float32)
    nb = nb / jnp.maximum(cnt, 1.0)

    nf3 = jnp.maximum(
        jnp.dot(nf_ref[...], wn_ref[...], precision=_HI,
                preferred_element_type=jnp.float32) + bn_ref[...], 0.0)
    nf3_ref[...] = nf3

    wd1 = wd_ref[0:D, :]
    wd2 = wd_ref[D:2 * D, :]
    bn_half = 0.5 * jnp.dot(nf3, wd2, precision=_HI,
                            preferred_element_type=jnp.float32)
    bnn_ref[...] = bn_half
    p_ref[...] = (jnp.dot(nb, wd1, precision=_HI,
                          preferred_element_type=jnp.float32)
                  + bn_half + bd_ref[...] + be_ref[...])
    wc_ref[...] = jnp.dot(we_ref[...], wd1, precision=_HI,
                          preferred_element_type=jnp.float32)


CH_FAST = 120
CH_SLOW = 40


def _gather_body(p_hbm, bn_hbm, dst_hbm, src_hbm, g_hbm,
                 idxd_v, idxs_v, bufp_v, bufb_v, semg):
    cid = lax.axis_index("c")
    sid = lax.axis_index("s")
    n_ch = jnp.where(cid == 0, CH_FAST, CH_SLOW)
    ch0 = jnp.where(cid == 0, sid * CH_FAST, NS * CH_FAST + sid * CH_SLOW)

    @pl.when(cid == 0)
    def _():
        pltpu.sync_copy(dst_hbm.at[pl.ds(sid * CH_FAST, CH_FAST)],
                        idxd_v.at[pl.ds(0, CH_FAST)])
        pltpu.sync_copy(src_hbm.at[pl.ds(sid * CH_FAST, CH_FAST)],
                        idxs_v.at[pl.ds(0, CH_FAST)])

    @pl.when(cid == 1)
    def _():
        b0 = NS * CH_FAST + sid * CH_SLOW
        pltpu.sync_copy(dst_hbm.at[pl.ds(b0, CH_SLOW)],
                        idxd_v.at[pl.ds(0, CH_SLOW)])
        pltpu.sync_copy(src_hbm.at[pl.ds(b0, CH_SLOW)],
                        idxs_v.at[pl.ds(0, CH_SLOW)])

    def start(b, slot):
        pltpu.async_copy(p_hbm.at[idxd_v.at[b]], bufp_v.at[slot], semg)
        pltpu.async_copy(bn_hbm.at[idxs_v.at[b]], bufb_v.at[slot], semg)

    def drain_write(b, slot):
        pltpu.make_async_copy(p_hbm.at[idxd_v.at[b]], bufp_v.at[slot],
                              semg).wait()
        pltpu.make_async_copy(bn_hbm.at[idxs_v.at[b]], bufb_v.at[slot],
                              semg).wait()

        def row_body(i, c):
            pr = bufp_v.at[slot, i]
            br = bufb_v.at[slot, i]
            for j in range(D // 16):
                s = pl.ds(j * 16, 16)
                pr[s] = pr[s] + br[s]
            return c

        lax.fori_loop(0, CHUNK, row_body, 0)
        pltpu.sync_copy(bufp_v.at[slot],
                        g_hbm.at[pl.ds((ch0 + b) * CHUNK, CHUNK)])

    start(0, 0)

    def step(g, carry):
        start(g + 1, 1)
        drain_write(g, 0)

        @pl.when(g + 2 < n_ch)
        def _():
            start(g + 2, 0)

        drain_write(g + 1, 1)
        return carry

    lax.fori_loop(0, n_ch // 2, lambda i, c: step(2 * i, c), 0)


_gather_kernel = functools.partial(
    pl.kernel,
    out_type=jax.ShapeDtypeStruct((E_PAD, D), jnp.float32),
    mesh=plsc.VectorSubcoreMesh(core_axis_name="c", subcore_axis_name="s"),
    scratch_types=[
        pltpu.VMEM((CH_FAST, CHUNK), jnp.int32),
        pltpu.VMEM((CH_FAST, CHUNK), jnp.int32),
        pltpu.VMEM((2, CHUNK, D), jnp.float32),
        pltpu.VMEM((2, CHUNK, D), jnp.float32),
        pltpu.SemaphoreType.DMA,
    ],
    compiler_params=pltpu.CompilerParams(use_tc_tiling_on_sc=False),
)


BLK_E = 3200


def _edge_body(ef_ref, wc_ref, g_ref, o_ref):
    o_ref[...] = jnp.maximum(
        jnp.dot(ef_ref[...], wc_ref[...], precision=_HI,
                preferred_element_type=jnp.float32)
        + g_ref[...], 0.0)


def kernel(nf, ef, edge_index, W_node, W_edge, bias_node, bias_edge,
           W_dense, b_dense):
    src = edge_index[0].astype(jnp.int32)
    dst = edge_index[1].astype(jnp.int32)
    pad = E_PAD - E
    dst2d = jnp.concatenate([dst, jnp.full((pad,), N, jnp.int32)]
                            ).reshape(E_PAD // CHUNK, CHUNK)
    src2d = jnp.concatenate([src, jnp.full((pad,), N, jnp.int32)]
                            ).reshape(E_PAD // CHUNK, CHUNK)
    ef_pad = jnp.concatenate([ef, jnp.zeros((pad, D_E), jnp.float32)], axis=0)
    ones_h = jnp.ones((CHUNK, D_E), jnp.float32)
    zeros_h = jnp.zeros((ROWS_PER_TILE, D_E), jnp.float32)

    psums, pcnts = _seg_kernel(_seg_body)(ef_pad, dst2d, ones_h, zeros_h)

    nf_pad = jnp.concatenate(
        [nf, jnp.zeros((N_PAD - N, D), jnp.float32)], axis=0)
    nrb = N_PAD // RB
    nf3_pad, P, Bn, W_c = pl.pallas_call(
        _node_body,
        grid=(nrb,),
        in_specs=[
            pl.BlockSpec((RB, D), lambda i: (i, 0)),
            pl.BlockSpec((D, D), lambda i: (0, 0)),
            pl.BlockSpec((1, D), lambda i: (0, 0)),
            pl.BlockSpec((NC, RB, D_E), lambda i: (0, i, 0)),
            pl.BlockSpec((NC, RB, D_E), lambda i: (0, i, 0)),
            pl.BlockSpec((D_E, D), lambda i: (0, 0)),
            pl.BlockSpec((2 * D, D), lambda i: (0, 0)),
            pl.BlockSpec((1, D), lambda i: (0, 0)),
            pl.BlockSpec((1, D), lambda i: (0, 0)),
        ],
        out_specs=[
            pl.BlockSpec((RB, D), lambda i: (i, 0)),
            pl.BlockSpec((RB, D), lambda i: (i, 0)),
            pl.BlockSpec((RB, D), lambda i: (i, 0)),
            pl.BlockSpec((D_E, D), lambda i: (0, 0)),
        ],
        out_shape=[
            jax.ShapeDtypeStruct((N_PAD, D), jnp.float32),
            jax.ShapeDtypeStruct((N_PAD, D), jnp.float32),
            jax.ShapeDtypeStruct((N_PAD, D), jnp.float32),
            jax.ShapeDtypeStruct((D_E, D), jnp.float32),
        ],
    )(nf_pad, W_node, bias_node.reshape(1, D), psums, pcnts,
      W_edge, W_dense, b_dense.reshape(1, D), bias_edge.reshape(1, D))
    nf3 = nf3_pad[:N]

    G = _gather_kernel(_gather_body)(P, Bn, dst2d, src2d)

    nblk = E // BLK_E
    ef3 = pl.pallas_call(
        _edge_body,
        grid=(nblk,),
        in_specs=[
            pl.BlockSpec((BLK_E, D_E), lambda i: (i, 0)),
            pl.BlockSpec((D_E, D), lambda i: (0, 0)),
            pl.BlockSpec((BLK_E, D), lambda i: (i, 0)),
        ],
        out_specs=pl.BlockSpec((BLK_E, D), lambda i: (i, 0)),
        out_shape=jax.ShapeDtypeStruct((E, D), jnp.float32),
    )(ef, W_c, G)

    return (nf3, ef3)


# speedup vs baseline: 1.0133x; 1.0133x over previous
import functools
import math

import jax
import jax.numpy as jnp
from jax import lax
from jax.experimental import pallas as pl
from jax.experimental.pallas import tpu as pltpu
from jax.experimental.pallas import tpu_sc as plsc

N = 10000
E = 320000
D_E = 16
D = 128

NC = 2
NS = 16
NW = NC * NS
CHUNK = 128
CH_PER_W = 80
E_PAD = NW * CH_PER_W * CHUNK
N_PAD = 10240
ROWS_PER_TILE = N_PAD // NS

_HI = jax.lax.Precision.HIGHEST


def _seg_body(ef_hbm, dst_hbm, ones_hbm, zeros_hbm,
              psums_hbm, pcnts_hbm,
              idx_v, efb_v, ones_v, zeros_v, tmp_v, sums_sp, cnts_sp, sem):
    cid = lax.axis_index("c")
    sid = lax.axis_index("s")
    wid = sid * NC + cid
    r0 = sid * ROWS_PER_TILE

    pltpu.sync_copy(zeros_hbm, zeros_v)
    pltpu.sync_copy(zeros_v, sums_sp.at[pl.ds(r0, ROWS_PER_TILE)])
    pltpu.sync_copy(zeros_v, cnts_sp.at[pl.ds(r0, ROWS_PER_TILE)])
    pltpu.sync_copy(ones_hbm, ones_v)
    pltpu.sync_copy(dst_hbm.at[pl.ds(wid * CH_PER_W, CH_PER_W)], idx_v)
    plsc.subcore_barrier()

    def step(b, carry):
        base = (wid * CH_PER_W + b) * CHUNK
        pltpu.sync_copy(ef_hbm.at[pl.ds(base, CHUNK)], efb_v)
        pltpu.sync_copy(efb_v, sums_sp.at[idx_v.at[b]], add=True)
        pltpu.sync_copy(ones_v, cnts_sp.at[idx_v.at[b]], add=True)
        return carry

    lax.fori_loop(0, CH_PER_W, step, 0)
    plsc.subcore_barrier()

    pltpu.sync_copy(sums_sp.at[pl.ds(r0, ROWS_PER_TILE)], tmp_v)
    pltpu.sync_copy(tmp_v, psums_hbm.at[cid].at[pl.ds(r0, ROWS_PER_TILE)])
    pltpu.sync_copy(cnts_sp.at[pl.ds(r0, ROWS_PER_TILE)], tmp_v)
    pltpu.sync_copy(tmp_v, pcnts_hbm.at[cid].at[pl.ds(r0, ROWS_PER_TILE)])


_seg_kernel = functools.partial(
    pl.kernel,
    out_type=[
        jax.ShapeDtypeStruct((NC, N_PAD, D_E), jnp.float32),
        jax.ShapeDtypeStruct((NC, N_PAD, D_E), jnp.float32),
    ],
    mesh=plsc.VectorSubcoreMesh(core_axis_name="c", subcore_axis_name="s"),
    scratch_types=[
        pltpu.VMEM((CH_PER_W, CHUNK), jnp.int32),
        pltpu.VMEM((CHUNK, D_E), jnp.float32),
        pltpu.VMEM((CHUNK, D_E), jnp.float32),
        pltpu.VMEM((ROWS_PER_TILE, D_E), jnp.float32),
        pltpu.VMEM((ROWS_PER_TILE, D_E), jnp.float32),
        pltpu.VMEM_SHARED((N_PAD, D_E), jnp.float32),
        pltpu.VMEM_SHARED((N_PAD, D_E), jnp.float32),
        pltpu.SemaphoreType.DMA,
    ],
    compiler_params=pltpu.CompilerParams(use_tc_tiling_on_sc=False),
)


RB = 1024


def _node_body(nf_ref, wn_ref, bn_ref, psums_ref, pcnts_ref,
               we_ref, wd_ref, bd_ref, be_ref,
               nf3_ref, p_ref, bnn_ref, wc_ref):
    sums = psums_ref[0] + psums_ref[1]
    cnt = pcnts_ref[0, :, 0:1] + pcnts_ref[1, :, 0:1]
    nb = jnp.dot(sums, we_ref[...], precision=_HI,
                 preferred_element_type=jnp.float32)
    nb = nb / jnp.maximum(cnt, 1.0)

    nf3 = jnp.maximum(
        jnp.dot(nf_ref[...], wn_ref[...], precision=_HI,
                preferred_element_type=jnp.float32) + bn_ref[...], 0.0)
    nf3_ref[...] = nf3

    wd1 = wd_ref[0:D, :]
    wd2 = wd_ref[D:2 * D, :]
    bn_half = 0.5 * jnp.dot(nf3, wd2, precision=_HI,
                            preferred_element_type=jnp.float32)
    bnn_ref[...] = bn_half
    p_ref[...] = (jnp.dot(nb, wd1, precision=_HI,
                          preferred_element_type=jnp.float32)
                  + bn_half + bd_ref[...] + be_ref[...])
    wc_ref[...] = jnp.dot(we_ref[...], wd1, precision=_HI,
                          preferred_element_type=jnp.float32)


CH_FAST = 120
CH_SLOW = 40


def _gather_body(p_hbm, bn_hbm, dst_hbm, src_hbm, g_hbm,
                 idxd_v, idxs_v, bufp_v, bufb_v, semg):
    cid = lax.axis_index("c")
    sid = lax.axis_index("s")
    n_ch = jnp.where(cid == 1, CH_FAST, CH_SLOW)
    ch0 = jnp.where(cid == 1, sid * CH_FAST, NS * CH_FAST + sid * CH_SLOW)

    @pl.when(cid == 1)
    def _():
        pltpu.sync_copy(dst_hbm.at[pl.ds(sid * CH_FAST, CH_FAST)],
                        idxd_v.at[pl.ds(0, CH_FAST)])
        pltpu.sync_copy(src_hbm.at[pl.ds(sid * CH_FAST, CH_FAST)],
                        idxs_v.at[pl.ds(0, CH_FAST)])

    @pl.when(cid == 0)
    def _():
        b0 = NS * CH_FAST + sid * CH_SLOW
        pltpu.sync_copy(dst_hbm.at[pl.ds(b0, CH_SLOW)],
                        idxd_v.at[pl.ds(0, CH_SLOW)])
        pltpu.sync_copy(src_hbm.at[pl.ds(b0, CH_SLOW)],
                        idxs_v.at[pl.ds(0, CH_SLOW)])

    def start(b, slot):
        pltpu.async_copy(p_hbm.at[idxd_v.at[b]], bufp_v.at[slot], semg)
        pltpu.async_copy(bn_hbm.at[idxs_v.at[b]], bufb_v.at[slot], semg)

    def drain_write(b, slot):
        pltpu.make_async_copy(p_hbm.at[idxd_v.at[b]], bufp_v.at[slot],
                              semg).wait()
        pltpu.make_async_copy(bn_hbm.at[idxs_v.at[b]], bufb_v.at[slot],
                              semg).wait()

        def row_body(i, c):
            pr = bufp_v.at[slot, i]
            br = bufb_v.at[slot, i]
            for j in range(D // 16):
                s = pl.ds(j * 16, 16)
                pr[s] = pr[s] + br[s]
            return c

        lax.fori_loop(0, CHUNK, row_body, 0)
        pltpu.sync_copy(bufp_v.at[slot],
                        g_hbm.at[pl.ds((ch0 + b) * CHUNK, CHUNK)])

    start(0, 0)

    def step(g, carry):
        start(g + 1, 1)
        drain_write(g, 0)

        @pl.when(g + 2 < n_ch)
        def _():
            start(g + 2, 0)

        drain_write(g + 1, 1)
        return carry

    lax.fori_loop(0, n_ch // 2, lambda i, c: step(2 * i, c), 0)


_gather_kernel = functools.partial(
    pl.kernel,
    out_type=jax.ShapeDtypeStruct((E_PAD, D), jnp.float32),
    mesh=plsc.VectorSubcoreMesh(core_axis_name="c", subcore_axis_name="s"),
    scratch_types=[
        pltpu.VMEM((CH_FAST, CHUNK), jnp.int32),
        pltpu.VMEM((CH_FAST, CHUNK), jnp.int32),
        pltpu.VMEM((2, CHUNK, D), jnp.float32),
        pltpu.VMEM((2, CHUNK, D), jnp.float32),
        pltpu.SemaphoreType.DMA,
    ],
    compiler_params=pltpu.CompilerParams(use_tc_tiling_on_sc=False),
)


BLK_E = 3200


def _edge_body(ef_ref, wc_ref, g_ref, o_ref):
    o_ref[...] = jnp.maximum(
        jnp.dot(ef_ref[...], wc_ref[...], precision=_HI,
                preferred_element_type=jnp.float32)
        + g_ref[...], 0.0)


def kernel(nf, ef, edge_index, W_node, W_edge, bias_node, bias_edge,
           W_dense, b_dense):
    src = edge_index[0].astype(jnp.int32)
    dst = edge_index[1].astype(jnp.int32)
    pad = E_PAD - E
    dst2d = jnp.concatenate([dst, jnp.full((pad,), N, jnp.int32)]
                            ).reshape(E_PAD // CHUNK, CHUNK)
    src2d = jnp.concatenate([src, jnp.full((pad,), N, jnp.int32)]
                            ).reshape(E_PAD // CHUNK, CHUNK)
    ef_pad = jnp.concatenate([ef, jnp.zeros((pad, D_E), jnp.float32)], axis=0)
    ones_h = jnp.ones((CHUNK, D_E), jnp.float32)
    zeros_h = jnp.zeros((ROWS_PER_TILE, D_E), jnp.float32)

    psums, pcnts = _seg_kernel(_seg_body)(ef_pad, dst2d, ones_h, zeros_h)

    nf_pad = jnp.concatenate(
        [nf, jnp.zeros((N_PAD - N, D), jnp.float32)], axis=0)
    nrb = N_PAD // RB
    nf3_pad, P, Bn, W_c = pl.pallas_call(
        _node_body,
        grid=(nrb,),
        in_specs=[
            pl.BlockSpec((RB, D), lambda i: (i, 0)),
            pl.BlockSpec((D, D), lambda i: (0, 0)),
            pl.BlockSpec((1, D), lambda i: (0, 0)),
            pl.BlockSpec((NC, RB, D_E), lambda i: (0, i, 0)),
            pl.BlockSpec((NC, RB, D_E), lambda i: (0, i, 0)),
            pl.BlockSpec((D_E, D), lambda i: (0, 0)),
            pl.BlockSpec((2 * D, D), lambda i: (0, 0)),
            pl.BlockSpec((1, D), lambda i: (0, 0)),
            pl.BlockSpec((1, D), lambda i: (0, 0)),
        ],
        out_specs=[
            pl.BlockSpec((RB, D), lambda i: (i, 0)),
            pl.BlockSpec((RB, D), lambda i: (i, 0)),
            pl.BlockSpec((RB, D), lambda i: (i, 0)),
            pl.BlockSpec((D_E, D), lambda i: (0, 0)),
        ],
        out_shape=[
            jax.ShapeDtypeStruct((N_PAD, D), jnp.float32),
            jax.ShapeDtypeStruct((N_PAD, D), jnp.float32),
            jax.ShapeDtypeStruct((N_PAD, D), jnp.float32),
            jax.ShapeDtypeStruct((D_E, D), jnp.float32),
        ],
    )(nf_pad, W_node, bias_node.reshape(1, D), psums, pcnts,
      W_edge, W_dense, b_dense.reshape(1, D), bias_edge.reshape(1, D))
    nf3 = nf3_pad[:N]

    G = _gather_kernel(_gather_body)(P, Bn, dst2d, src2d)

    nblk = E // BLK_E
    ef3 = pl.pallas_call(
        _edge_body,
        grid=(nblk,),
        in_specs=[
            pl.BlockSpec((BLK_E, D_E), lambda i: (i, 0)),
            pl.BlockSpec((D_E, D), lambda i: (0, 0)),
            pl.BlockSpec((BLK_E, D), lambda i: (i, 0)),
        ],
        out_specs=pl.BlockSpec((BLK_E, D), lambda i: (i, 0)),
        out_shape=jax.ShapeDtypeStruct((E, D), jnp.float32),
    )(ef, W_c, G)

    return (nf3, ef3)


# speedup vs baseline: 1.2073x; 1.1914x over previous
import functools

import jax
import jax.numpy as jnp
from jax import lax
from jax.experimental import pallas as pl
from jax.experimental.pallas import tpu as pltpu
from jax.experimental.pallas import tpu_sc as plsc

N = 10000
E = 320000
D_E = 16
D = 128

NC = 2
NS = 16
NW = NC * NS
CHUNK = 128
CH_PER_W = 80
N_CH = NW * CH_PER_W
E_PAD = N_CH * CHUNK
N_REAL_CH = E // CHUNK
N_PAD = 10240
FR = 32

_HI = jax.lax.Precision.HIGHEST


def _seg_body(eft_hbm, dst_hbm, ones_hbm, zeros_hbm, pacc_hbm,
              idx_v, efb_v, ones_v, tmp_v, acc_sp, sem):
    cid = lax.axis_index("c")
    sid = lax.axis_index("s")
    wid = sid * NC + cid

    pltpu.sync_copy(zeros_hbm, tmp_v)
    pltpu.sync_copy(tmp_v, acc_sp.at[pl.ds(2 * sid, 2)])
    pltpu.sync_copy(ones_hbm, ones_v)
    pltpu.sync_copy(dst_hbm.at[pl.ds(wid * CH_PER_W, CH_PER_W)], idx_v)
    plsc.subcore_barrier()

    def step(b, carry):
        ch = wid * CH_PER_W + b
        base = jnp.where(ch < N_REAL_CH, ch * CHUNK, 0)
        pltpu.sync_copy(eft_hbm.at[:, pl.ds(base, CHUNK)], efb_v)
        for f in range(D_E):
            pltpu.sync_copy(efb_v.at[f], acc_sp.at[f].at[idx_v.at[b]],
                            add=True)
        pltpu.sync_copy(ones_v, acc_sp.at[D_E].at[idx_v.at[b]], add=True)
        return carry

    lax.fori_loop(0, CH_PER_W, step, 0)
    plsc.subcore_barrier()

    pltpu.sync_copy(acc_sp.at[pl.ds(2 * sid, 2)], tmp_v)
    pltpu.sync_copy(tmp_v, pacc_hbm.at[cid].at[pl.ds(2 * sid, 2)])


_seg_kernel = functools.partial(
    pl.kernel,
    out_type=jax.ShapeDtypeStruct((NC, FR, N_PAD), jnp.float32),
    mesh=plsc.VectorSubcoreMesh(core_axis_name="c", subcore_axis_name="s"),
    scratch_types=[
        pltpu.VMEM((CH_PER_W, CHUNK), jnp.int32),
        pltpu.VMEM((D_E, CHUNK), jnp.float32),
        pltpu.VMEM((CHUNK,), jnp.float32),
        pltpu.VMEM((2, N_PAD), jnp.float32),
        pltpu.VMEM_SHARED((FR, N_PAD), jnp.float32),
        pltpu.SemaphoreType.DMA,
    ],
    compiler_params=pltpu.CompilerParams(use_tc_tiling_on_sc=False),
)


RB = 1024


def _node_body(nf_ref, wn_ref, bn_ref, pacc_ref,
               we_ref, wd_ref, bd_ref, be_ref,
               nf3_ref, p_ref, bnn_ref, wc_ref):
    wd1 = wd_ref[0:D, :]
    wd2 = wd_ref[D:2 * D, :]
    wc = jnp.dot(we_ref[...], wd1, precision=_HI,
                 preferred_element_type=jnp.float32)
    wc_ref[...] = wc

    sums_t = pacc_ref[0, :D_E, :] + pacc_ref[1, :D_E, :]
    cnt = pacc_ref[0, D_E:D_E + 1, :] + pacc_ref[1, D_E:D_E + 1, :]
    recip = 1.0 / jnp.maximum(cnt, 1.0)
    nb_div = lax.dot_general(sums_t * recip, wc,
                             (((0,), (0,)), ((), ())), precision=_HI,
                             preferred_element_type=jnp.float32)

    nf3 = jnp.maximum(
        jnp.dot(nf_ref[...], wn_ref[...], precision=_HI,
                preferred_element_type=jnp.float32) + bn_ref[...], 0.0)
    nf3_ref[...] = nf3

    bn_half = 0.5 * jnp.dot(nf3, wd2, precision=_HI,
                            preferred_element_type=jnp.float32)
    bnn_ref[...] = bn_half
    p_ref[...] = nb_div + bn_half + bd_ref[...] + be_ref[...]


def _gather_body(p_hbm, bn_hbm, dst_hbm, src_hbm, g_hbm,
                 idxd_v, idxs_v, bufp_v, bufb_v, semg):
    cid = lax.axis_index("c")
    sid = lax.axis_index("s")
    ch0 = (sid * NC + cid) * CH_PER_W
    pltpu.sync_copy(dst_hbm.at[pl.ds(ch0, CH_PER_W)], idxd_v)
    pltpu.sync_copy(src_hbm.at[pl.ds(ch0, CH_PER_W)], idxs_v)

    def start(b, slot):
        pltpu.async_copy(p_hbm.at[idxd_v.at[b]], bufp_v.at[slot], semg)
        pltpu.async_copy(bn_hbm.at[idxs_v.at[b]], bufb_v.at[slot], semg)

    def drain_write(b, slot):
        pltpu.make_async_copy(p_hbm.at[idxd_v.at[b]], bufp_v.at[slot],
                              semg).wait()
        pltpu.make_async_copy(bn_hbm.at[idxs_v.at[b]], bufb_v.at[slot],
                              semg).wait()

        def row_body(i, c):
            pr = bufp_v.at[slot, i]
            br = bufb_v.at[slot, i]
            for j in range(D // 16):
                s = pl.ds(j * 16, 16)
                pr[s] = pr[s] + br[s]
            return c

        lax.fori_loop(0, CHUNK, row_body, 0)
        pltpu.sync_copy(bufp_v.at[slot],
                        g_hbm.at[pl.ds((ch0 + b) * CHUNK, CHUNK)])

    start(0, 0)

    def step(g, carry):
        start(g + 1, 1)
        drain_write(g, 0)

        @pl.when(g + 2 < CH_PER_W)
        def _():
            start(g + 2, 0)

        drain_write(g + 1, 1)
        return carry

    lax.fori_loop(0, CH_PER_W // 2, lambda i, c: step(2 * i, c), 0)


_gather_kernel = functools.partial(
    pl.kernel,
    out_type=jax.ShapeDtypeStruct((E_PAD, D), jnp.float32),
    mesh=plsc.VectorSubcoreMesh(core_axis_name="c", subcore_axis_name="s"),
    scratch_types=[
        pltpu.VMEM((CH_PER_W, CHUNK), jnp.int32),
        pltpu.VMEM((CH_PER_W, CHUNK), jnp.int32),
        pltpu.VMEM((2, CHUNK, D), jnp.float32),
        pltpu.VMEM((2, CHUNK, D), jnp.float32),
        pltpu.SemaphoreType.DMA,
    ],
    compiler_params=pltpu.CompilerParams(use_tc_tiling_on_sc=False),
)


BLK_E = 3200


def _edge_body(eft_ref, wc_ref, g_ref, o_ref):
    o_ref[...] = jnp.maximum(
        lax.dot_general(eft_ref[...], wc_ref[...],
                        (((0,), (0,)), ((), ())), precision=_HI,
                        preferred_element_type=jnp.float32)
        + g_ref[...], 0.0)


def kernel(nf, ef, edge_index, W_node, W_edge, bias_node, bias_edge,
           W_dense, b_dense):
    src = edge_index[0].astype(jnp.int32)
    dst = edge_index[1].astype(jnp.int32)
    pad = E_PAD - E
    dst2d = jnp.concatenate([dst, jnp.full((pad,), N, jnp.int32)]
                            ).reshape(N_CH, CHUNK)
    src2d = jnp.concatenate([src, jnp.full((pad,), N, jnp.int32)]
                            ).reshape(N_CH, CHUNK)
    eft = ef.T
    ones_h = jnp.ones((CHUNK,), jnp.float32)
    zeros_h = jnp.zeros((2, N_PAD), jnp.float32)

    pacc = _seg_kernel(_seg_body)(eft, dst2d, ones_h, zeros_h)

    nf_pad = jnp.concatenate(
        [nf, jnp.zeros((N_PAD - N, D), jnp.float32)], axis=0)
    nrb = N_PAD // RB
    nf3_pad, P, Bn, W_c = pl.pallas_call(
        _node_body,
        grid=(nrb,),
        in_specs=[
            pl.BlockSpec((RB, D), lambda i: (i, 0)),
            pl.BlockSpec((D, D), lambda i: (0, 0)),
            pl.BlockSpec((1, D), lambda i: (0, 0)),
            pl.BlockSpec((NC, FR, RB), lambda i: (0, 0, i)),
            pl.BlockSpec((D_E, D), lambda i: (0, 0)),
            pl.BlockSpec((2 * D, D), lambda i: (0, 0)),
            pl.BlockSpec((1, D), lambda i: (0, 0)),
            pl.BlockSpec((1, D), lambda i: (0, 0)),
        ],
        out_specs=[
            pl.BlockSpec((RB, D), lambda i: (i, 0)),
            pl.BlockSpec((RB, D), lambda i: (i, 0)),
            pl.BlockSpec((RB, D), lambda i: (i, 0)),
            pl.BlockSpec((D_E, D), lambda i: (0, 0)),
        ],
        out_shape=[
            jax.ShapeDtypeStruct((N_PAD, D), jnp.float32),
            jax.ShapeDtypeStruct((N_PAD, D), jnp.float32),
            jax.ShapeDtypeStruct((N_PAD, D), jnp.float32),
            jax.ShapeDtypeStruct((D_E, D), jnp.float32),
        ],
    )(nf_pad, W_node, bias_node.reshape(1, D), pacc,
      W_edge, W_dense, b_dense.reshape(1, D), bias_edge.reshape(1, D))
    nf3 = nf3_pad[:N]

    G = _gather_kernel(_gather_body)(P, Bn, dst2d, src2d)

    nblk = E // BLK_E
    ef3 = pl.pallas_call(
        _edge_body,
        grid=(nblk,),
        in_specs=[
            pl.BlockSpec((D_E, BLK_E), lambda i: (0, i)),
            pl.BlockSpec((D_E, D), lambda i: (0, 0)),
            pl.BlockSpec((BLK_E, D), lambda i: (i, 0)),
        ],
        out_specs=pl.BlockSpec((BLK_E, D), lambda i: (i, 0)),
        out_shape=jax.ShapeDtypeStruct((E, D), jnp.float32),
    )(eft, W_c, G)

    return (nf3, ef3)


# speedup vs baseline: 1.2795x; 1.0597x over previous
import functools

import jax
import jax.numpy as jnp
from jax import lax
from jax.experimental import pallas as pl
from jax.experimental.pallas import tpu as pltpu
from jax.experimental.pallas import tpu_sc as plsc

N = 10000
E = 320000
D_E = 16
D = 128

NC = 2
NS = 16
NW = NC * NS
CHUNK = 128
CH_PER_W = 80
N_CH = NW * CH_PER_W
E_PAD = N_CH * CHUNK
N_REAL_CH = E // CHUNK
N_PAD = 10240
FR = 32

_HI = jax.lax.Precision.HIGHEST


def _seg_body(eft_hbm, dst_hbm, ones_hbm, zeros_hbm, pacc_hbm,
              idx_v, efb_v, ones_v, tmp_v, acc_sp, sem_e, sem_s):
    cid = lax.axis_index("c")
    sid = lax.axis_index("s")
    wid = sid * NC + cid

    pltpu.sync_copy(zeros_hbm, tmp_v)
    pltpu.sync_copy(tmp_v, acc_sp.at[pl.ds(2 * sid, 2)])
    pltpu.sync_copy(ones_hbm, ones_v)
    pltpu.sync_copy(dst_hbm.at[pl.ds(wid * CH_PER_W, CH_PER_W)], idx_v)
    plsc.subcore_barrier()

    def base(b):
        ch = wid * CH_PER_W + b
        return jnp.where(ch < N_REAL_CH, ch * CHUNK, 0)

    def fire_load(b, slot):
        pltpu.async_copy(eft_hbm.at[:, pl.ds(base(b), CHUNK)],
                         efb_v.at[slot], sem_e)

    def wait_load(b, slot):
        pltpu.make_async_copy(eft_hbm.at[:, pl.ds(base(b), CHUNK)],
                              efb_v.at[slot], sem_e).wait()

    def scat(b, slot):
        for f in range(D_E):
            pltpu.async_copy(efb_v.at[slot, f],
                             acc_sp.at[f].at[idx_v.at[b]], sem_s, add=True)
        pltpu.async_copy(ones_v, acc_sp.at[D_E].at[idx_v.at[b]],
                         sem_s, add=True)

    def drain_scat(b, slot):
        for f in range(D_E):
            pltpu.make_async_copy(efb_v.at[slot, f],
                                  acc_sp.at[f].at[idx_v.at[b]], sem_s).wait()
        pltpu.make_async_copy(ones_v, acc_sp.at[D_E].at[idx_v.at[b]],
                              sem_s).wait()

    for k in range(3):
        fire_load(k, k)

    def chunk(b, j):
        wait_load(b, j)
        scat(b, j)

        @pl.when(b >= 1)
        def _():
            drain_scat(b - 1, (j + 3) % 4)

        @pl.when(b + 3 < CH_PER_W)
        def _():
            fire_load(b + 3, (j + 3) % 4)

    def quad(q, carry):
        for j in range(4):
            chunk(4 * q + j, j)
        return carry

    lax.fori_loop(0, CH_PER_W // 4, quad, 0)
    drain_scat(CH_PER_W - 1, (CH_PER_W - 1) % 4)
    plsc.subcore_barrier()

    pltpu.sync_copy(acc_sp.at[pl.ds(2 * sid, 2)], tmp_v)
    pltpu.sync_copy(tmp_v, pacc_hbm.at[cid].at[pl.ds(2 * sid, 2)])


_seg_kernel = functools.partial(
    pl.kernel,
    out_type=jax.ShapeDtypeStruct((NC, FR, N_PAD), jnp.float32),
    mesh=plsc.VectorSubcoreMesh(core_axis_name="c", subcore_axis_name="s"),
    scratch_types=[
        pltpu.VMEM((CH_PER_W, CHUNK), jnp.int32),
        pltpu.VMEM((4, D_E, CHUNK), jnp.float32),
        pltpu.VMEM((CHUNK,), jnp.float32),
        pltpu.VMEM((2, N_PAD), jnp.float32),
        pltpu.VMEM_SHARED((FR, N_PAD), jnp.float32),
        pltpu.SemaphoreType.DMA,
        pltpu.SemaphoreType.DMA,
    ],
    compiler_params=pltpu.CompilerParams(use_tc_tiling_on_sc=False),
)


RB = 1024


def _node_body(nf_ref, wn_ref, bn_ref, pacc_ref,
               we_ref, wd_ref, bd_ref, be_ref,
               nf3_ref, p_ref, bnn_ref, wc_ref):
    wd1 = wd_ref[0:D, :]
    wd2 = wd_ref[D:2 * D, :]
    wc = jnp.dot(we_ref[...], wd1, precision=_HI,
                 preferred_element_type=jnp.float32)
    wc_ref[...] = wc

    sums_t = pacc_ref[0, :D_E, :] + pacc_ref[1, :D_E, :]
    cnt = pacc_ref[0, D_E:D_E + 1, :] + pacc_ref[1, D_E:D_E + 1, :]
    recip = 1.0 / jnp.maximum(cnt, 1.0)
    nb_div = lax.dot_general(sums_t * recip, wc,
                             (((0,), (0,)), ((), ())), precision=_HI,
                             preferred_element_type=jnp.float32)

    nf3 = jnp.maximum(
        jnp.dot(nf_ref[...], wn_ref[...], precision=_HI,
                preferred_element_type=jnp.float32) + bn_ref[...], 0.0)
    nf3_ref[...] = nf3

    bn_half = 0.5 * jnp.dot(nf3, wd2, precision=_HI,
                            preferred_element_type=jnp.float32)
    bnn_ref[...] = bn_half
    p_ref[...] = nb_div + bn_half + bd_ref[...] + be_ref[...]


CH_C0 = 114
CH_C1 = 46


def _gather_run(p_hbm, bn_hbm, dst_hbm, src_hbm, g_hbm,
                idxd_v, idxs_v, bufp_v, bufb_v, semg, ch0, n_ch):
    pltpu.sync_copy(dst_hbm.at[pl.ds(ch0, n_ch)], idxd_v.at[pl.ds(0, n_ch)])
    pltpu.sync_copy(src_hbm.at[pl.ds(ch0, n_ch)], idxs_v.at[pl.ds(0, n_ch)])

    def start(b, slot):
        pltpu.async_copy(p_hbm.at[idxd_v.at[b]], bufp_v.at[slot], semg)
        pltpu.async_copy(bn_hbm.at[idxs_v.at[b]], bufb_v.at[slot], semg)

    def drain_write(b, slot):
        pltpu.make_async_copy(p_hbm.at[idxd_v.at[b]], bufp_v.at[slot],
                              semg).wait()
        pltpu.make_async_copy(bn_hbm.at[idxs_v.at[b]], bufb_v.at[slot],
                              semg).wait()

        def row_body(i, c):
            pr = bufp_v.at[slot, i]
            br = bufb_v.at[slot, i]
            for j in range(D // 16):
                s = pl.ds(j * 16, 16)
                pr[s] = pr[s] + br[s]
            return c

        lax.fori_loop(0, CHUNK, row_body, 0)
        pltpu.sync_copy(bufp_v.at[slot],
                        g_hbm.at[pl.ds((ch0 + b) * CHUNK, CHUNK)])

    start(0, 0)

    def step(g, carry):
        start(g + 1, 1)
        drain_write(g, 0)

        @pl.when(g + 2 < n_ch)
        def _():
            start(g + 2, 0)

        drain_write(g + 1, 1)
        return carry

    lax.fori_loop(0, n_ch // 2, lambda i, c: step(2 * i, c), 0)


def _gather_body(p_hbm, bn_hbm, dst_hbm, src_hbm, g_hbm,
                 idxd_v, idxs_v, bufp_v, bufb_v, semg):
    cid = lax.axis_index("c")
    sid = lax.axis_index("s")
    args = (p_hbm, bn_hbm, dst_hbm, src_hbm, g_hbm,
            idxd_v, idxs_v, bufp_v, bufb_v, semg)

    @pl.when(cid == 0)
    def _():
        _gather_run(*args, sid * CH_C0, CH_C0)

    @pl.when(cid == 1)
    def _():
        _gather_run(*args, NS * CH_C0 + sid * CH_C1, CH_C1)


_gather_kernel = functools.partial(
    pl.kernel,
    out_type=jax.ShapeDtypeStruct((E_PAD, D), jnp.float32),
    mesh=plsc.VectorSubcoreMesh(core_axis_name="c", subcore_axis_name="s"),
    scratch_types=[
        pltpu.VMEM((CH_C0, CHUNK), jnp.int32),
        pltpu.VMEM((CH_C0, CHUNK), jnp.int32),
        pltpu.VMEM((2, CHUNK, D), jnp.float32),
        pltpu.VMEM((2, CHUNK, D), jnp.float32),
        pltpu.SemaphoreType.DMA,
    ],
    compiler_params=pltpu.CompilerParams(use_tc_tiling_on_sc=False),
)


BLK_E = 6400


def _edge_body(eft_ref, wc_ref, g_ref, o_ref):
    o_ref[...] = jnp.maximum(
        lax.dot_general(eft_ref[...], wc_ref[...],
                        (((0,), (0,)), ((), ())), precision=_HI,
                        preferred_element_type=jnp.float32)
        + g_ref[...], 0.0)


def kernel(nf, ef, edge_index, W_node, W_edge, bias_node, bias_edge,
           W_dense, b_dense):
    src = edge_index[0].astype(jnp.int32)
    dst = edge_index[1].astype(jnp.int32)
    pad = E_PAD - E
    dst2d = jnp.concatenate([dst, jnp.full((pad,), N, jnp.int32)]
                            ).reshape(N_CH, CHUNK)
    src2d = jnp.concatenate([src, jnp.full((pad,), N, jnp.int32)]
                            ).reshape(N_CH, CHUNK)
    eft = ef.T
    ones_h = jnp.ones((CHUNK,), jnp.float32)
    zeros_h = jnp.zeros((2, N_PAD), jnp.float32)

    pacc = _seg_kernel(_seg_body)(eft, dst2d, ones_h, zeros_h)

    nf_pad = jnp.concatenate(
        [nf, jnp.zeros((N_PAD - N, D), jnp.float32)], axis=0)
    nrb = N_PAD // RB
    nf3_pad, P, Bn, W_c = pl.pallas_call(
        _node_body,
        grid=(nrb,),
        in_specs=[
            pl.BlockSpec((RB, D), lambda i: (i, 0)),
            pl.BlockSpec((D, D), lambda i: (0, 0)),
            pl.BlockSpec((1, D), lambda i: (0, 0)),
            pl.BlockSpec((NC, FR, RB), lambda i: (0, 0, i)),
            pl.BlockSpec((D_E, D), lambda i: (0, 0)),
            pl.BlockSpec((2 * D, D), lambda i: (0, 0)),
            pl.BlockSpec((1, D), lambda i: (0, 0)),
            pl.BlockSpec((1, D), lambda i: (0, 0)),
        ],
        out_specs=[
            pl.BlockSpec((RB, D), lambda i: (i, 0)),
            pl.BlockSpec((RB, D), lambda i: (i, 0)),
            pl.BlockSpec((RB, D), lambda i: (i, 0)),
            pl.BlockSpec((D_E, D), lambda i: (0, 0)),
        ],
        out_shape=[
            jax.ShapeDtypeStruct((N_PAD, D), jnp.float32),
            jax.ShapeDtypeStruct((N_PAD, D), jnp.float32),
            jax.ShapeDtypeStruct((N_PAD, D), jnp.float32),
            jax.ShapeDtypeStruct((D_E, D), jnp.float32),
        ],
    )(nf_pad, W_node, bias_node.reshape(1, D), pacc,
      W_edge, W_dense, b_dense.reshape(1, D), bias_edge.reshape(1, D))
    nf3 = nf3_pad[:N]

    G = _gather_kernel(_gather_body)(P, Bn, dst2d, src2d)

    nblk = E // BLK_E
    ef3 = pl.pallas_call(
        _edge_body,
        grid=(nblk,),
        in_specs=[
            pl.BlockSpec((D_E, BLK_E), lambda i: (0, i)),
            pl.BlockSpec((D_E, D), lambda i: (0, 0)),
            pl.BlockSpec((BLK_E, D), lambda i: (i, 0)),
        ],
        out_specs=pl.BlockSpec((BLK_E, D), lambda i: (i, 0)),
        out_shape=jax.ShapeDtypeStruct((E, D), jnp.float32),
    )(eft, W_c, G)

    return (nf3, ef3)


# speedup vs baseline: 1.3782x; 1.0772x over previous
import functools

import jax
import jax.numpy as jnp
from jax import lax
from jax.experimental import pallas as pl
from jax.experimental.pallas import tpu as pltpu
from jax.experimental.pallas import tpu_sc as plsc

N = 10000
E = 320000
D_E = 16
D = 128

NC = 2
NS = 16
NW = NC * NS
CHUNK = 128
CH_PER_W = 80
N_CH = NW * CH_PER_W
E_PAD = N_CH * CHUNK
N_REAL_CH = E // CHUNK
N_PAD = 10240
FR = 32

_HI = jax.lax.Precision.HIGHEST


def _seg_body(eft_hbm, dst_hbm, ones_hbm, zeros_hbm, pacc_hbm,
              idx_v, efb_v, ones_v, tmp_v, acc_sp, sem_e, sem_s):
    cid = lax.axis_index("c")
    sid = lax.axis_index("s")
    wid = sid * NC + cid

    pltpu.sync_copy(zeros_hbm, tmp_v)
    pltpu.sync_copy(tmp_v, acc_sp.at[pl.ds(2 * sid, 2)])
    pltpu.sync_copy(ones_hbm, ones_v)
    pltpu.sync_copy(dst_hbm.at[pl.ds(wid * CH_PER_W, CH_PER_W)], idx_v)
    plsc.subcore_barrier()

    def base(b):
        ch = wid * CH_PER_W + b
        return jnp.where(ch < N_REAL_CH, ch * CHUNK, 0)

    def fire_load(b, slot):
        pltpu.async_copy(eft_hbm.at[:, pl.ds(base(b), CHUNK)],
                         efb_v.at[slot], sem_e)

    def wait_load(b, slot):
        pltpu.make_async_copy(eft_hbm.at[:, pl.ds(base(b), CHUNK)],
                              efb_v.at[slot], sem_e).wait()

    def scat(b, slot):
        for f in range(D_E):
            pltpu.async_copy(efb_v.at[slot, f],
                             acc_sp.at[f].at[idx_v.at[b]], sem_s, add=True)
        pltpu.async_copy(ones_v, acc_sp.at[D_E].at[idx_v.at[b]],
                         sem_s, add=True)

    def drain_scat(b, slot):
        for f in range(D_E):
            pltpu.make_async_copy(efb_v.at[slot, f],
                                  acc_sp.at[f].at[idx_v.at[b]], sem_s).wait()
        pltpu.make_async_copy(ones_v, acc_sp.at[D_E].at[idx_v.at[b]],
                              sem_s).wait()

    for k in range(3):
        fire_load(k, k)

    def chunk(b, j):
        wait_load(b, j)
        scat(b, j)

        @pl.when(b >= 1)
        def _():
            drain_scat(b - 1, (j + 3) % 4)

        @pl.when(b + 3 < CH_PER_W)
        def _():
            fire_load(b + 3, (j + 3) % 4)

    def quad(q, carry):
        for j in range(4):
            chunk(4 * q + j, j)
        return carry

    lax.fori_loop(0, CH_PER_W // 4, quad, 0)
    drain_scat(CH_PER_W - 1, (CH_PER_W - 1) % 4)
    plsc.subcore_barrier()

    pltpu.sync_copy(acc_sp.at[pl.ds(2 * sid, 2)], tmp_v)
    pltpu.sync_copy(tmp_v, pacc_hbm.at[cid].at[pl.ds(2 * sid, 2)])


_seg_kernel = functools.partial(
    pl.kernel,
    out_type=jax.ShapeDtypeStruct((NC, FR, N_PAD), jnp.float32),
    mesh=plsc.VectorSubcoreMesh(core_axis_name="c", subcore_axis_name="s"),
    scratch_types=[
        pltpu.VMEM((CH_PER_W, CHUNK), jnp.int32),
        pltpu.VMEM((4, D_E, CHUNK), jnp.float32),
        pltpu.VMEM((CHUNK,), jnp.float32),
        pltpu.VMEM((2, N_PAD), jnp.float32),
        pltpu.VMEM_SHARED((FR, N_PAD), jnp.float32),
        pltpu.SemaphoreType.DMA,
        pltpu.SemaphoreType.DMA,
    ],
    compiler_params=pltpu.CompilerParams(use_tc_tiling_on_sc=False),
)


RB = 1024


def _node_body(nf_ref, wn_ref, bn_ref, pacc_ref,
               we_ref, wd_ref, bd_ref, be_ref,
               nf3_ref, p_ref, bnn_ref, wc_ref):
    wd1 = wd_ref[0:D, :]
    wd2 = wd_ref[D:2 * D, :]
    wc = jnp.dot(we_ref[...], wd1, precision=_HI,
                 preferred_element_type=jnp.float32)
    wc_ref[...] = wc

    sums_t = pacc_ref[0, :D_E, :] + pacc_ref[1, :D_E, :]
    cnt = pacc_ref[0, D_E:D_E + 1, :] + pacc_ref[1, D_E:D_E + 1, :]
    recip = 1.0 / jnp.maximum(cnt, 1.0)
    nb_div = lax.dot_general(sums_t * recip, wc,
                             (((0,), (0,)), ((), ())), precision=_HI,
                             preferred_element_type=jnp.float32)

    nf3 = jnp.maximum(
        jnp.dot(nf_ref[...], wn_ref[...], precision=_HI,
                preferred_element_type=jnp.float32) + bn_ref[...], 0.0)
    nf3_ref[...] = nf3

    bn_half = 0.5 * jnp.dot(nf3, wd2, precision=_HI,
                            preferred_element_type=jnp.float32)
    bnn_ref[...] = bn_half
    p_ref[...] = nb_div + bn_half + bd_ref[...] + be_ref[...]


HALF_CH = N_CH // 2
CH_HALF_W = CH_PER_W // 2


def _gather_run(p_hbm, bn_hbm, dst_hbm, src_hbm, g_hbm,
                idxd_v, idxs_v, bufp_v, bufb_v, semg, ch0, n_ch, row0):
    pltpu.sync_copy(dst_hbm.at[pl.ds(ch0, n_ch)], idxd_v.at[pl.ds(0, n_ch)])
    pltpu.sync_copy(src_hbm.at[pl.ds(ch0, n_ch)], idxs_v.at[pl.ds(0, n_ch)])

    def start(b, slot):
        pltpu.async_copy(p_hbm.at[idxd_v.at[b]], bufp_v.at[slot], semg)
        pltpu.async_copy(bn_hbm.at[idxs_v.at[b]], bufb_v.at[slot], semg)

    def drain_write(b, slot):
        pltpu.make_async_copy(p_hbm.at[idxd_v.at[b]], bufp_v.at[slot],
                              semg).wait()
        pltpu.make_async_copy(bn_hbm.at[idxs_v.at[b]], bufb_v.at[slot],
                              semg).wait()

        def row_body(i, c):
            pr = bufp_v.at[slot, i]
            br = bufb_v.at[slot, i]
            for j in range(D // 16):
                s = pl.ds(j * 16, 16)
                pr[s] = pr[s] + br[s]
            return c

        lax.fori_loop(0, CHUNK, row_body, 0)
        pltpu.sync_copy(bufp_v.at[slot],
                        g_hbm.at[pl.ds(row0 + b * CHUNK, CHUNK)])

    start(0, 0)

    def step(g, carry):
        start(g + 1, 1)
        drain_write(g, 0)

        @pl.when(g + 2 < n_ch)
        def _():
            start(g + 2, 0)

        drain_write(g + 1, 1)
        return carry

    lax.fori_loop(0, n_ch // 2, lambda i, c: step(2 * i, c), 0)


def _make_gather_body(half):
    def body(p_hbm, bn_hbm, dst_hbm, src_hbm, g_hbm,
             idxd_v, idxs_v, bufp_v, bufb_v, semg):
        cid = lax.axis_index("c")
        sid = lax.axis_index("s")
        wid = sid * NC + cid
        _gather_run(p_hbm, bn_hbm, dst_hbm, src_hbm, g_hbm,
                    idxd_v, idxs_v, bufp_v, bufb_v, semg,
                    half * HALF_CH + wid * CH_HALF_W, CH_HALF_W,
                    wid * CH_HALF_W * CHUNK)
    return body


_gather_kernel = functools.partial(
    pl.kernel,
    out_type=jax.ShapeDtypeStruct((E_PAD // 2, D), jnp.float32),
    mesh=plsc.VectorSubcoreMesh(core_axis_name="c", subcore_axis_name="s"),
    scratch_types=[
        pltpu.VMEM((CH_HALF_W, CHUNK), jnp.int32),
        pltpu.VMEM((CH_HALF_W, CHUNK), jnp.int32),
        pltpu.VMEM((2, CHUNK, D), jnp.float32),
        pltpu.VMEM((2, CHUNK, D), jnp.float32),
        pltpu.SemaphoreType.DMA,
    ],
    compiler_params=pltpu.CompilerParams(use_tc_tiling_on_sc=False),
)


BLK_E = 2560


def _edge_body(eft_ref, wc_ref, g_ref, o_ref):
    o_ref[...] = jnp.maximum(
        lax.dot_general(eft_ref[...], wc_ref[...],
                        (((0,), (0,)), ((), ())), precision=_HI,
                        preferred_element_type=jnp.float32)
        + g_ref[...], 0.0)


def _edge_body2(eft_ref, wc_ref, g_ref, prev_ref, o_ref):
    o_ref[...] = jnp.maximum(
        lax.dot_general(eft_ref[...], wc_ref[...],
                        (((0,), (0,)), ((), ())), precision=_HI,
                        preferred_element_type=jnp.float32)
        + g_ref[...], 0.0)


def kernel(nf, ef, edge_index, W_node, W_edge, bias_node, bias_edge,
           W_dense, b_dense):
    src = edge_index[0].astype(jnp.int32)
    dst = edge_index[1].astype(jnp.int32)
    pad = E_PAD - E
    dst2d = jnp.concatenate([dst, jnp.full((pad,), N, jnp.int32)]
                            ).reshape(N_CH, CHUNK)
    src2d = jnp.concatenate([src, jnp.full((pad,), N, jnp.int32)]
                            ).reshape(N_CH, CHUNK)
    eft = ef.T
    ones_h = jnp.ones((CHUNK,), jnp.float32)
    zeros_h = jnp.zeros((2, N_PAD), jnp.float32)

    pacc = _seg_kernel(_seg_body)(eft, dst2d, ones_h, zeros_h)

    nf_pad = jnp.concatenate(
        [nf, jnp.zeros((N_PAD - N, D), jnp.float32)], axis=0)
    nrb = N_PAD // RB
    nf3_pad, P, Bn, W_c = pl.pallas_call(
        _node_body,
        grid=(nrb,),
        in_specs=[
            pl.BlockSpec((RB, D), lambda i: (i, 0)),
            pl.BlockSpec((D, D), lambda i: (0, 0)),
            pl.BlockSpec((1, D), lambda i: (0, 0)),
            pl.BlockSpec((NC, FR, RB), lambda i: (0, 0, i)),
            pl.BlockSpec((D_E, D), lambda i: (0, 0)),
            pl.BlockSpec((2 * D, D), lambda i: (0, 0)),
            pl.BlockSpec((1, D), lambda i: (0, 0)),
            pl.BlockSpec((1, D), lambda i: (0, 0)),
        ],
        out_specs=[
            pl.BlockSpec((RB, D), lambda i: (i, 0)),
            pl.BlockSpec((RB, D), lambda i: (i, 0)),
            pl.BlockSpec((RB, D), lambda i: (i, 0)),
            pl.BlockSpec((D_E, D), lambda i: (0, 0)),
        ],
        out_shape=[
            jax.ShapeDtypeStruct((N_PAD, D), jnp.float32),
            jax.ShapeDtypeStruct((N_PAD, D), jnp.float32),
            jax.ShapeDtypeStruct((N_PAD, D), jnp.float32),
            jax.ShapeDtypeStruct((D_E, D), jnp.float32),
        ],
    )(nf_pad, W_node, bias_node.reshape(1, D), pacc,
      W_edge, W_dense, b_dense.reshape(1, D), bias_edge.reshape(1, D))
    nf3 = nf3_pad[:N]

    G1 = _gather_kernel(_make_gather_body(0))(P, Bn, dst2d, src2d)
    G2 = _gather_kernel(_make_gather_body(1))(P, Bn, dst2d, src2d)

    nblk1 = (E_PAD // 2) // BLK_E
    nblk2 = (E - E_PAD // 2) // BLK_E
    ef3_a = pl.pallas_call(
        _edge_body,
        grid=(nblk1,),
        in_specs=[
            pl.BlockSpec((D_E, BLK_E), lambda i: (0, i)),
            pl.BlockSpec((D_E, D), lambda i: (0, 0)),
            pl.BlockSpec((BLK_E, D), lambda i: (i, 0)),
        ],
        out_specs=pl.BlockSpec((BLK_E, D), lambda i: (i, 0)),
        out_shape=jax.ShapeDtypeStruct((E, D), jnp.float32),
    )(eft, W_c, G1)
    ef3 = pl.pallas_call(
        _edge_body2,
        grid=(nblk2,),
        in_specs=[
            pl.BlockSpec((D_E, BLK_E), lambda i: (0, i + nblk1)),
            pl.BlockSpec((D_E, D), lambda i: (0, 0)),
            pl.BlockSpec((BLK_E, D), lambda i: (i, 0)),
            pl.BlockSpec(memory_space=pl.ANY),
        ],
        out_specs=pl.BlockSpec((BLK_E, D), lambda i: (i + nblk1, 0)),
        out_shape=jax.ShapeDtypeStruct((E, D), jnp.float32),
        input_output_aliases={3: 0},
    )(eft, W_c, G2, ef3_a)

    return (nf3, ef3)
